# baseline (speedup 1.0000x reference)
import numpy as np
import jax
import jax.numpy as jnp
from jax.experimental import pallas as pl
from jax.experimental.pallas import tpu as pltpu

_MASK_RATIO = 0.5
_B, _T, _P, _D = 8, 16, 576, 768
_N = _B * _T

_MASK = np.asarray(jax.random.uniform(jax.random.key(42), (_B, _T)) < _MASK_RATIO)
_FLAT = _MASK.ravel()

_RUNS = []
_s = 0
for _i in range(1, _N + 1):
    if _i == _N or _FLAT[_i] != _FLAT[_s]:
        _RUNS.append((_s, _i - _s, bool(_FLAT[_s])))
        _s = _i
_MASKED_IDS = np.nonzero(_FLAT)[0]


def _body(x_ref, tok_ref, out_ref, tokf, sem_cp, sem_tok):
    tokf[...] = jnp.broadcast_to(tok_ref[0, :], (_P, _D))
    copies = []
    for start, length, masked in _RUNS:
        if not masked:
            c = pltpu.make_async_copy(
                x_ref.at[pl.ds(start, length)],
                out_ref.at[pl.ds(start, length)],
                sem_cp,
            )
            c.start()
            copies.append(c)
    tok_copies = []
    for f in _MASKED_IDS:
        c = pltpu.make_async_copy(tokf, out_ref.at[int(f)], sem_tok)
        c.start()
        tok_copies.append(c)
    for c in copies:
        c.wait()
    for c in tok_copies:
        c.wait()


def kernel(x, mask_token):
    x3 = x.reshape(_N, _P, _D)
    tok = mask_token.reshape(1, _D)
    out3 = pl.pallas_call(
        _body,
        in_specs=[
            pl.BlockSpec(memory_space=pl.ANY),
            pl.BlockSpec(memory_space=pltpu.VMEM),
        ],
        out_specs=pl.BlockSpec(memory_space=pl.ANY),
        out_shape=jax.ShapeDtypeStruct((_N, _P, _D), x.dtype),
        scratch_shapes=[
            pltpu.VMEM((_P, _D), jnp.float32),
            pltpu.SemaphoreType.DMA,
            pltpu.SemaphoreType.DMA,
        ],
    )(x3, tok)
    return out3.reshape(_B, _T, _P, _D)


# speedup vs baseline: 23.4113x; 23.4113x over previous
import numpy as np
import jax
import jax.numpy as jnp
from jax.experimental import pallas as pl
from jax.experimental.pallas import tpu as pltpu

_MASK_RATIO = 0.5
_B, _T, _P, _D = 8, 16, 576, 768
_N = _B * _T

_MASK = np.asarray(jax.random.uniform(jax.random.key(42), (_B, _T)) < _MASK_RATIO)
_FLAT = _MASK.ravel().astype(np.int32)

_SRC = np.empty(_N, np.int32)
_prev = int(np.nonzero(_FLAT == 0)[0][0]) if (_FLAT == 0).any() else 0
for _i in range(_N):
    if not _FLAT[_i]:
        _prev = _i
    _SRC[_i] = _prev


def _body(src_ref, flag_ref, x_ref, tok_ref, out_ref):
    i = pl.program_id(0)
    tok = tok_ref[0, :]
    out_ref[0] = jnp.where(flag_ref[i] != 0, tok[None, :], x_ref[0])


def kernel(x, mask_token):
    x3 = x.reshape(_N, _P, _D)
    tok = mask_token.reshape(1, _D)
    grid_spec = pltpu.PrefetchScalarGridSpec(
        num_scalar_prefetch=2,
        grid=(_N,),
        in_specs=[
            pl.BlockSpec((1, _P, _D), lambda i, src, flag: (src[i], 0, 0)),
            pl.BlockSpec((1, _D), lambda i, src, flag: (0, 0)),
        ],
        out_specs=pl.BlockSpec((1, _P, _D), lambda i, src, flag: (i, 0, 0)),
    )
    out3 = pl.pallas_call(
        _body,
        grid_spec=grid_spec,
        out_shape=jax.ShapeDtypeStruct((_N, _P, _D), x.dtype),
    )(jnp.asarray(_SRC), jnp.asarray(_FLAT), x3, tok)
    return out3.reshape(_B, _T, _P, _D)


# speedup vs baseline: 27.9694x; 1.1947x over previous
import numpy as np
import jax
import jax.numpy as jnp
from jax.experimental import pallas as pl
from jax.experimental.pallas import tpu as pltpu

_MASK_RATIO = 0.5
_B, _T, _P, _D = 8, 16, 576, 768
_N = _B * _T

_MASK = np.asarray(jax.random.uniform(jax.random.key(42), (_B, _T)) < _MASK_RATIO)
_MASKED = np.nonzero(_MASK.ravel())[0].astype(np.int32)
_UNMASKED = np.nonzero(~_MASK.ravel())[0].astype(np.int32)
_M = int(_MASKED.size)

_PIN = int(_UNMASKED[0]) if _UNMASKED.size else 0
_SRC = np.concatenate([np.full(_M, _PIN, np.int32), _UNMASKED]).astype(np.int32)
_DST = np.concatenate([_MASKED, _UNMASKED]).astype(np.int32)


def _body(src_ref, dst_ref, x_ref, tok_ref, out_ref):
    i = pl.program_id(0)

    @pl.when(i < _M)
    def _():
        out_ref[0] = jnp.broadcast_to(tok_ref[0, :], (_P, _D))

    @pl.when(i >= _M)
    def _():
        out_ref[0] = x_ref[0]


def kernel(x, mask_token):
    x3 = x.reshape(_N, _P, _D)
    tok = mask_token.reshape(1, _D)
    grid_spec = pltpu.PrefetchScalarGridSpec(
        num_scalar_prefetch=2,
        grid=(_N,),
        in_specs=[
            pl.BlockSpec((1, _P, _D), lambda i, src, dst: (src[i], 0, 0)),
            pl.BlockSpec((1, _D), lambda i, src, dst: (0, 0)),
        ],
        out_specs=pl.BlockSpec((1, _P, _D), lambda i, src, dst: (dst[i], 0, 0)),
    )
    out3 = pl.pallas_call(
        _body,
        grid_spec=grid_spec,
        out_shape=jax.ShapeDtypeStruct((_N, _P, _D), x.dtype),
    )(jnp.asarray(_SRC), jnp.asarray(_DST), x3, tok)
    return out3.reshape(_B, _T, _P, _D)
